# baseline (speedup 1.0000x reference)
import functools

import jax
import jax.numpy as jnp
from jax import lax
from jax.experimental import pallas as pl
from jax.experimental.pallas import tpu as pltpu
from jax.experimental.pallas import tpu_sc as plsc

BATCH = 16384
EMB = 64
PLOT_DIM = 384
WIN = 128
NBUF = 4


def _make_sc_gather(B, b_per_w, v_user, v_movie):
    info = plsc.get_sparse_core_info()
    NC, NS = info.num_cores, info.num_subcores
    mesh = plsc.VectorSubcoreMesh(core_axis_name="c", subcore_axis_name="s")

    @functools.partial(
        pl.kernel,
        mesh=mesh,
        compiler_params=pltpu.CompilerParams(
            disable_bounds_checks=True, needs_layout_passes=False),
        out_type=(
            jax.ShapeDtypeStruct((B, 2 * EMB), jnp.float32),
            jax.ShapeDtypeStruct((B, 2 * EMB), jnp.float32),
        ),
        scratch_types=[
            pltpu.VMEM((b_per_w,), jnp.int32),
            pltpu.VMEM((b_per_w,), jnp.int32),
            pltpu.VMEM((b_per_w,), jnp.int32),
            pltpu.VMEM((b_per_w,), jnp.int32),
            pltpu.VMEM((16,), jnp.int32),
            pltpu.VMEM((EMB, WIN), jnp.float32),
            pltpu.VMEM((EMB, WIN), jnp.float32),
            pltpu.VMEM((EMB, WIN), jnp.float32),
            pltpu.VMEM((EMB, WIN), jnp.float32),
            pltpu.VMEM((b_per_w, 2 * EMB), jnp.float32),
            pltpu.SemaphoreType.DMA,
            pltpu.SemaphoreType.DMA,
            pltpu.SemaphoreType.DMA,
            pltpu.SemaphoreType.DMA,
            pltpu.SemaphoreType.DMA,
        ],
    )
    def gather_kernel(utabT, us, uperm, uchg, upf, uwl,
                      mtabT, ms, mperm, mchg, mpf, mwl, uout, mout,
                      idx_v, perm_v, chg_v, pf_v, head_v,
                      win0, win1, win2, win3,
                      rows_v, sem0, sem1, sem2, sem3, osem):
        wid = lax.axis_index("s") * NC + lax.axis_index("c")
        base = wid * b_per_w
        iota = lax.iota(jnp.int32, 16)
        cvecs = [iota + q * 16 for q in range(4)]
        wins = [win0, win1, win2, win3]
        sems = [sem0, sem1, sem2, sem3]

        def gather_sorted(tabT, sidx, perm, chg, pf, wl, out, wmax):
            pltpu.sync_copy(sidx.at[pl.ds(base, b_per_w)], idx_v)
            pltpu.sync_copy(perm.at[pl.ds(base, b_per_w)], perm_v)
            pltpu.sync_copy(chg.at[pl.ds(base, b_per_w)], chg_v)
            pltpu.sync_copy(pf.at[pl.ds(base, b_per_w)], pf_v)
            pltpu.sync_copy(wl.at[pl.ds(base, 16)], head_v)

            def fetch(wf, b):
                wc = lax.min(lax.max(wf, 0), wmax)
                off = pl.multiple_of(wc * WIN, WIN)
                pltpu.async_copy(tabT.at[:, pl.ds(off, WIN)], wins[b], sems[b])

            head = head_v[pl.ds(0, 16)]
            for p in range(NBUF - 1):
                fetch(head[p], p)

            def body(g, kc):
                vec_r = idx_v[pl.ds(g * 16, 16)]
                vec_c = chg_v[pl.ds(g * 16, 16)]
                vec_p = pf_v[pl.ds(g * 16, 16)]
                for j in range(16):
                    r = vec_r[j]
                    kn = kc + vec_c[j]

                    @pl.when(vec_c[j] == 1)
                    def _():
                        for b in range(NBUF):
                            @pl.when((kn & (NBUF - 1)) == b)
                            def _(b=b):
                                pltpu.make_async_copy(
                                    tabT.at[:, pl.ds(0, WIN)], wins[b],
                                    sems[b]).wait()
                        for b in range(NBUF):
                            @pl.when(((kn + NBUF - 1) & (NBUF - 1)) == b)
                            def _(b=b):
                                fetch(vec_p[j], b)

                    i = g * 16 + j
                    colv = jnp.full((16,), lax.bitwise_and(r, WIN - 1),
                                    jnp.int32)
                    for b in range(NBUF):
                        @pl.when((kn & (NBUF - 1)) == b)
                        def _(b=b):
                            for q in range(4):
                                vals = plsc.load_gather(
                                    wins[b], [cvecs[q], colv])
                                rows_v[i, pl.ds(q * 16, 16)] = vals
                    kc = kn
                return kc

            kf = lax.fori_loop(0, b_per_w // 16, body, jnp.int32(-1))
            for b in range(NBUF):
                @pl.when((kf & (NBUF - 1)) != b)
                def _(b=b):
                    pltpu.make_async_copy(
                        tabT.at[:, pl.ds(0, WIN)], wins[b], sems[b]).wait()
            pltpu.async_copy(rows_v, out.at[perm_v], osem).wait()

        gather_sorted(mtabT, ms, mperm, mchg, mpf, mwl, mout,
                      (v_movie - 1) >> 7)
        gather_sorted(utabT, us, uperm, uchg, upf, uwl, uout,
                      (v_user - 1) >> 7)

    return gather_kernel


def _mlp_body(u_ref, m_ref, p_ref, w1u_ref, w1m_ref, w1p_ref, b1_ref,
              w2_ref, b2_ref, w3_ref, b3_ref, w4_ref, b4_ref, o_ref):
    u = u_ref[...][:, :EMB]
    m = m_ref[...][:, :EMB]
    x = (jnp.dot(u, w1u_ref[...], preferred_element_type=jnp.float32)
         + jnp.dot(m, w1m_ref[...], preferred_element_type=jnp.float32)
         + jnp.dot(p_ref[...], w1p_ref[...], preferred_element_type=jnp.float32)
         + b1_ref[...])
    x = jnp.maximum(x, 0.0)
    x = jnp.maximum(
        jnp.dot(x, w2_ref[...], preferred_element_type=jnp.float32) + b2_ref[...], 0.0)
    x = jnp.maximum(
        jnp.dot(x, w3_ref[...], preferred_element_type=jnp.float32) + b3_ref[...], 0.0)
    o_ref[...] = jnp.dot(x, w4_ref[...], preferred_element_type=jnp.float32) + b4_ref[...]


def _mlp(urows, mrows, plot, W1u, W1m, W1p, b1, W2, b2, W3, b3, W4, b4,
         block_rows):
    B = urows.shape[0]
    grid = (B // block_rows,)

    def rows(i):
        return (i, 0)

    def whole(i):
        return (0, 0)

    return pl.pallas_call(
        _mlp_body,
        grid=grid,
        in_specs=[
            pl.BlockSpec((block_rows, 2 * EMB), rows),
            pl.BlockSpec((block_rows, 2 * EMB), rows),
            pl.BlockSpec((block_rows, PLOT_DIM), rows),
            pl.BlockSpec(W1u.shape, whole),
            pl.BlockSpec(W1m.shape, whole),
            pl.BlockSpec(W1p.shape, whole),
            pl.BlockSpec(b1.shape, whole),
            pl.BlockSpec(W2.shape, whole),
            pl.BlockSpec(b2.shape, whole),
            pl.BlockSpec(W3.shape, whole),
            pl.BlockSpec(b3.shape, whole),
            pl.BlockSpec(W4.shape, whole),
            pl.BlockSpec(b4.shape, whole),
        ],
        out_specs=pl.BlockSpec((block_rows, 1), rows),
        out_shape=jax.ShapeDtypeStruct((B, 1), jnp.float32),
    )(urows, mrows, plot, W1u, W1m, W1p, b1, W2, b2, W3, b3, W4, b4)


def _prep(idx32, b_per_w):
    B = idx32.shape[0]
    i_ar = lax.iota(jnp.int32, B)
    s, perm = lax.sort_key_val(idx32, i_ar)
    w = lax.shift_right_logical(s, 7)
    segfirst = (i_ar % b_per_w) == 0
    prev = jnp.concatenate([w[:1] - 1, w[:-1]])
    changed = jnp.logical_or(w != prev, segfirst)
    cs = jnp.cumsum(changed.astype(jnp.int32))
    segbase = cs[(i_ar // b_per_w) * b_per_w]
    slot = cs - segbase
    pos = (i_ar // b_per_w) * b_per_w + slot
    wlist = jnp.zeros((B,), jnp.int32).at[pos].set(w, mode="drop")
    pf_pos = ((i_ar // b_per_w) * b_per_w
              + jnp.minimum(slot + NBUF - 1, b_per_w - 1))
    pf = wlist[pf_pos]
    return s, perm, changed.astype(jnp.int32), pf, wlist


def kernel(users, movies, plot_embeddings, user_table, movie_table,
           W1, b1, W2, b2, W3, b3, W4, b4):
    info = plsc.get_sparse_core_info()
    b_per_w = BATCH // (info.num_cores * info.num_subcores)
    us, uperm, uchg, upf, uwl = _prep(users.astype(jnp.int32), b_per_w)
    ms, mperm, mchg, mpf, mwl = _prep(movies.astype(jnp.int32), b_per_w)
    urows, mrows = _make_sc_gather(
        BATCH, b_per_w, user_table.shape[0], movie_table.shape[0])(
        user_table.T, us, uperm, uchg, upf, uwl,
        movie_table.T, ms, mperm, mchg, mpf, mwl)
    W1u = W1[:EMB]
    W1m = W1[EMB:2 * EMB]
    W1p = W1[2 * EMB:]
    return _mlp(urows, mrows, plot_embeddings,
                W1u, W1m, W1p, b1.reshape(1, -1),
                W2, b2.reshape(1, -1), W3, b3.reshape(1, -1),
                W4, b4.reshape(1, -1), block_rows=2048)


# speedup vs baseline: 1.3172x; 1.3172x over previous
import functools

import jax
import jax.numpy as jnp
from jax import lax
from jax.experimental import pallas as pl
from jax.experimental.pallas import tpu as pltpu
from jax.experimental.pallas import tpu_sc as plsc

BATCH = 16384
EMB = 64
PLOT_DIM = 384
WIN = 128


def _make_sc_gather(B):
    info = plsc.get_sparse_core_info()
    NC, NS = info.num_cores, info.num_subcores
    NW = NC * NS
    b_per_w = B // NW
    mesh = plsc.VectorSubcoreMesh(core_axis_name="c", subcore_axis_name="s")

    @functools.partial(
        pl.kernel,
        mesh=mesh,
        compiler_params=pltpu.CompilerParams(
            disable_bounds_checks=True, needs_layout_passes=False),
        out_type=(
            jax.ShapeDtypeStruct((B, 2 * EMB), jnp.float32),
            jax.ShapeDtypeStruct((B, 2 * EMB), jnp.float32),
        ),
        scratch_types=[
            pltpu.VMEM((b_per_w,), jnp.int32),
            pltpu.VMEM((b_per_w,), jnp.int32),
            pltpu.VMEM((EMB, WIN), jnp.float32),
            pltpu.VMEM((EMB, WIN), jnp.float32),
            pltpu.VMEM((EMB, WIN), jnp.float32),
            pltpu.VMEM((EMB, WIN), jnp.float32),
            pltpu.VMEM((b_per_w, 2 * EMB), jnp.float32),
            pltpu.SemaphoreType.DMA,
            pltpu.SemaphoreType.DMA,
            pltpu.SemaphoreType.DMA,
            pltpu.SemaphoreType.DMA,
            pltpu.SemaphoreType.DMA,
        ],
    )
    def gather_kernel(utabT, us, uperm, mtabT, ms, mperm, uout, mout,
                      idx_v, perm_v, win0, win1, win2, win3, rows_v,
                      sem0, sem1, sem2, sem3, osem):
        wid = lax.axis_index("s") * NC + lax.axis_index("c")
        base = wid * b_per_w
        iota = lax.iota(jnp.int32, 16)
        cvecs = [iota + q * 16 for q in range(4)]
        wins = [win0, win1, win2, win3]
        sems = [sem0, sem1, sem2, sem3]
        NWALK = 4
        wpw = b_per_w // NWALK

        def gather_sorted(tabT, sidx, perm, out):
            pltpu.sync_copy(sidx.at[pl.ds(base, b_per_w)], idx_v)
            pltpu.sync_copy(perm.at[pl.ds(base, b_per_w)], perm_v)

            def body(g, carry):
                vecs = [idx_v[pl.ds(s * wpw + g * 16, 16)]
                        for s in range(NWALK)]
                for j in range(16):
                    rs = [vecs[s][j] for s in range(NWALK)]
                    wn = [lax.shift_right_logical(r, 7) for r in rs]
                    for s in range(NWALK):
                        @pl.when(wn[s] != carry[s])
                        def _(s=s):
                            off = pl.multiple_of(wn[s] * WIN, WIN)
                            pltpu.async_copy(
                                tabT.at[:, pl.ds(off, WIN)], wins[s], sems[s])
                    for s in range(NWALK):
                        @pl.when(wn[s] != carry[s])
                        def _(s=s):
                            pltpu.make_async_copy(
                                tabT.at[:, pl.ds(0, WIN)], wins[s],
                                sems[s]).wait()
                        i = s * wpw + g * 16 + j
                        colv = jnp.full((16,), lax.bitwise_and(rs[s], WIN - 1),
                                        jnp.int32)
                        for q in range(4):
                            vals = plsc.load_gather(wins[s], [cvecs[q], colv])
                            rows_v[i, pl.ds(q * 16, 16)] = vals
                    carry = tuple(wn)
                return carry

            lax.fori_loop(0, wpw // 16, body, (jnp.int32(-1),) * NWALK)
            pltpu.async_copy(rows_v, out.at[perm_v], osem).wait()

        gather_sorted(mtabT, ms, mperm, mout)
        gather_sorted(utabT, us, uperm, uout)

    return gather_kernel


def _mlp_body(u_ref, m_ref, p_ref, w1u_ref, w1m_ref, w1p_ref, b1_ref,
              w2_ref, b2_ref, w3_ref, b3_ref, w4_ref, b4_ref, o_ref):
    u = u_ref[...][:, :EMB]
    m = m_ref[...][:, :EMB]
    x = (jnp.dot(u, w1u_ref[...], preferred_element_type=jnp.float32)
         + jnp.dot(m, w1m_ref[...], preferred_element_type=jnp.float32)
         + jnp.dot(p_ref[...], w1p_ref[...], preferred_element_type=jnp.float32)
         + b1_ref[...])
    x = jnp.maximum(x, 0.0)
    x = jnp.maximum(
        jnp.dot(x, w2_ref[...], preferred_element_type=jnp.float32) + b2_ref[...], 0.0)
    x = jnp.maximum(
        jnp.dot(x, w3_ref[...], preferred_element_type=jnp.float32) + b3_ref[...], 0.0)
    o_ref[...] = jnp.dot(x, w4_ref[...], preferred_element_type=jnp.float32) + b4_ref[...]


def _mlp(urows, mrows, plot, W1u, W1m, W1p, b1, W2, b2, W3, b3, W4, b4,
         block_rows):
    B = urows.shape[0]
    grid = (B // block_rows,)

    def rows(i):
        return (i, 0)

    def whole(i):
        return (0, 0)

    return pl.pallas_call(
        _mlp_body,
        grid=grid,
        in_specs=[
            pl.BlockSpec((block_rows, 2 * EMB), rows),
            pl.BlockSpec((block_rows, 2 * EMB), rows),
            pl.BlockSpec((block_rows, PLOT_DIM), rows),
            pl.BlockSpec(W1u.shape, whole),
            pl.BlockSpec(W1m.shape, whole),
            pl.BlockSpec(W1p.shape, whole),
            pl.BlockSpec(b1.shape, whole),
            pl.BlockSpec(W2.shape, whole),
            pl.BlockSpec(b2.shape, whole),
            pl.BlockSpec(W3.shape, whole),
            pl.BlockSpec(b3.shape, whole),
            pl.BlockSpec(W4.shape, whole),
            pl.BlockSpec(b4.shape, whole),
        ],
        out_specs=pl.BlockSpec((block_rows, 1), rows),
        out_shape=jax.ShapeDtypeStruct((B, 1), jnp.float32),
    )(urows, mrows, plot, W1u, W1m, W1p, b1, W2, b2, W3, b3, W4, b4)


def kernel(users, movies, plot_embeddings, user_table, movie_table,
           W1, b1, W2, b2, W3, b3, W4, b4):
    u32 = users.astype(jnp.int32)
    m32 = movies.astype(jnp.int32)
    OFF = jnp.int32(1 << 20)
    cat = jnp.concatenate([u32, m32 + OFF])
    skeys, sperm = lax.sort_key_val(cat, lax.iota(jnp.int32, 2 * BATCH))
    us = skeys[:BATCH]
    uperm = sperm[:BATCH]
    ms = skeys[BATCH:] - OFF
    mperm = sperm[BATCH:] - BATCH
    urows, mrows = _make_sc_gather(BATCH)(
        user_table.T, us, uperm, movie_table.T, ms, mperm)
    W1u = W1[:EMB]
    W1m = W1[EMB:2 * EMB]
    W1p = W1[2 * EMB:]
    return _mlp(urows, mrows, plot_embeddings,
                W1u, W1m, W1p, b1.reshape(1, -1),
                W2, b2.reshape(1, -1), W3, b3.reshape(1, -1),
                W4, b4.reshape(1, -1), block_rows=4096)
